# baseline (speedup 1.0000x reference)
import jax
import jax.numpy as jnp
from jax import lax
from jax.experimental import pallas as pl
from jax.experimental.pallas import tpu as pltpu

_B = 16384
_N = 6
_M = 20
_BN = 16384


def _tc_body(w_ref, ft_ref, out_ref):
    out_ref[...] = lax.dot_general(
        w_ref[...], ft_ref[...], (((0,), (0,)), ((), ())),
        preferred_element_type=jnp.float32,
    )


def kernel(f_values, W):
    out_t = pl.pallas_call(
        _tc_body,
        grid=(_B // _BN,),
        in_specs=[
            pl.BlockSpec((_N, _M), lambda i: (0, 0)),
            pl.BlockSpec((_N, _BN), lambda i: (0, i)),
        ],
        out_specs=pl.BlockSpec((_M, _BN), lambda i: (0, i)),
        out_shape=jax.ShapeDtypeStruct((_M, _B), jnp.float32),
    )(W, f_values.T)
    return out_t.T
